# baseline (speedup 1.0000x reference)
import math

import jax
import jax.numpy as jnp
from jax import lax
from jax.experimental import pallas as pl
from jax.experimental.pallas import tpu as pltpu
from jax.experimental.pallas import tpu_sc as plsc

B, H, W = 8, 32, 32
HW = H * W
TMAX = HW
SQW = math.sqrt(W)
NC, NS = 1, 16
L = 16
CPR = W // L
SQT = 1928


def _astar_body(cm_hbm, sqt_hbm, gm_hbm, sm_hbm, hist_hbm, path_hbm,
                cm_v, sqt_v, gm_v, sm_v, val_v, g_v, open_v, hist_v,
                parents_v, path_v, sem1, sem2, sem3, sem4):
    wid = lax.axis_index("s") * NC + lax.axis_index("c")

    @pl.when(wid < B)
    def _():
        b = wid
        c1 = pltpu.async_copy(cm_hbm.at[b, 0], cm_v, sem1)
        c2 = pltpu.async_copy(sqt_hbm, sqt_v, sem2)
        c3 = pltpu.async_copy(gm_hbm.at[b, 0], gm_v, sem3)
        c4 = pltpu.async_copy(sm_hbm.at[b, 0], sm_v, sem4)

        lane = lax.iota(jnp.int32, L)
        zero_f = jnp.zeros((L,), jnp.float32)
        one_f = jnp.ones((L,), jnp.float32)
        one_i = jnp.ones((L,), jnp.int32)
        lane0 = lane == 0

        c3.wait()
        c4.wait()

        def find_chunk(c, acc):
            r = c >> 1
            half = (c & 1) * L
            gmc = gm_v[r, pl.ds(half, L)]
            smc = sm_v[r, pl.ds(half, L)]
            negidx = -(c * L + lane).astype(jnp.float32)
            gacc, sacc = acc
            gcand = jnp.max(jnp.where(gmc > 0.5, negidx, -float(HW)))
            scand = jnp.max(jnp.where(smc > 0.5, negidx, -float(HW)))
            return jnp.maximum(gacc, gcand), jnp.maximum(sacc, scand)

        gneg, sneg = lax.fori_loop(0, HW // L, find_chunk,
                                   (jnp.float32(-HW), jnp.float32(-HW)))
        goal_i = (-gneg).astype(jnp.int32)
        start_i = (-sneg).astype(jnp.int32)

        goal_fill = jnp.full((L,), goal_i, jnp.int32)
        zero_i = jnp.zeros((L,), jnp.int32)

        def init_chunk(c, _):
            s = c * L
            g_v[pl.ds(s, L)] = zero_f
            open_v[pl.ds(s, L)] = zero_f
            val_v[pl.ds(s, L)] = zero_f
            parents_v[pl.ds(s, L)] = goal_fill
            hist_v[c >> 1, pl.ds((c & 1) * L, L)] = zero_f
            path_v[c >> 1, pl.ds((c & 1) * L, L)] = zero_i
            return 0

        lax.fori_loop(0, HW // L, init_chunk, 0)

        startv = jnp.full((L,), start_i, jnp.int32)
        goalv = jnp.full((L,), goal_i, jnp.int32)
        gyv = goalv >> 5
        gxv = goalv & 31
        c2.wait()

        def h_at(yv, xv):
            dyv = jnp.abs(yv - gyv)
            dxv = jnp.abs(xv - gxv)
            euc = plsc.load_gather(sqt_v, [dyv * dyv + dxv * dxv])
            cheb = jnp.maximum(dyv, dxv).astype(jnp.float32)
            return cheb + 0.001 * euc

        hs = h_at(startv >> 5, startv & 31)
        plsc.store_scatter(open_v, [startv], one_f, mask=lane0)
        plsc.store_scatter(val_v, [startv], jnp.exp(-(0.5 * hs) / SQW),
                           mask=lane0)
        plsc.store_scatter(path_v, [gyv, gxv], one_i, mask=lane0)
        c1.wait()

        nk = lane + (lane >= 4).astype(jnp.int32)
        dy = nk // 3 - 1
        dx = nk % 3 - 1
        nbr_lane = lane < 8

        def cond(carry):
            t, solved = carry[0], carry[1]
            return jnp.logical_and(t < TMAX, jnp.logical_not(solved))

        def body(carry):
            t, _, t1, ymin, ymax = carry

            def acond(st):
                return st[0] < (ymax + 1) * CPR

            def achunk(st):
                c, bv, bi = st
                s = c * L
                v = val_v[pl.ds(s, L)]
                m = v > bv
                return (c + 1, jnp.where(m, v, bv),
                        jnp.where(m, s + lane, bi))

            _, bv, bi = lax.while_loop(
                acond, achunk,
                (ymin * CPR, jnp.full((L,), -1.0, jnp.float32),
                 jnp.zeros((L,), jnp.int32)))
            maxv = jnp.max(bv)
            ind = (-jnp.max(jnp.where(bv == maxv, -bi.astype(jnp.float32),
                                      -float(HW)))).astype(jnp.int32)
            indv = jnp.full((L,), ind, jnp.int32)
            solved = ind == goal_i

            iyv = indv >> 5
            ixv = indv & 31
            plsc.store_scatter(hist_v, [iyv, ixv], one_f, mask=lane0)
            rm = jnp.logical_and(lane0, jnp.logical_not(solved))
            plsc.store_scatter(open_v, [indv], zero_f, mask=rm)
            plsc.store_scatter(val_v, [indv], zero_f, mask=rm)

            new_gv = plsc.load_gather(g_v, [indv]) + plsc.load_gather(
                cm_v, [iyv, ixv])

            iy = ind >> 5
            ny = iyv + dy
            nx = ixv + dx
            valid = (nbr_lane & (ny >= 0) & (ny < H) & (nx >= 0) & (nx < W))
            nyc = jnp.where(valid, ny, 0)
            nxc = jnp.where(valid, nx, 0)
            nidx = nyc * W + nxc
            gn = plsc.load_gather(g_v, [nidx], mask=valid)
            on = plsc.load_gather(open_v, [nidx], mask=valid)
            hn = plsc.load_gather(hist_v, [nyc, nxc], mask=valid)
            hh = h_at(nyc, nxc)
            upd = valid & (((on <= 0.5) & (hn <= 0.5))
                           | ((on > 0.5) & (gn > new_gv)))
            plsc.store_scatter(g_v, [nidx], new_gv, mask=upd)
            plsc.store_scatter(open_v, [nidx], one_f, mask=upd)
            plsc.store_scatter(parents_v, [nidx], indv, mask=upd)
            fn = 0.5 * new_gv + 0.5 * hh
            plsc.store_scatter(val_v, [nidx], jnp.exp(-fn / SQW), mask=upd)

            t1 = jnp.where(solved, t, t1)
            ymin = jnp.minimum(ymin, jnp.maximum(iy - 1, 0))
            ymax = jnp.maximum(ymax, jnp.minimum(iy + 1, H - 1))
            return t + 1, solved, t1, ymin, ymax

        sy = start_i >> 5
        _, _, t1, _, _ = lax.while_loop(
            cond, body,
            (jnp.int32(0), jnp.bool_(False), jnp.int32(TMAX - 1), sy, sy))

        loc0 = jnp.max(plsc.load_gather(parents_v,
                                        [goalv]).astype(jnp.float32))

        def bt(_, loc):
            locv = jnp.full((L,), loc.astype(jnp.int32), jnp.int32)
            plsc.store_scatter(path_v, [locv >> 5, locv & 31], one_i,
                               mask=lane0)
            return jnp.max(plsc.load_gather(parents_v,
                                            [locv]).astype(jnp.float32))

        lax.fori_loop(0, t1, bt, loc0)

        o1 = pltpu.async_copy(hist_v, hist_hbm.at[b, 0], sem1)
        o2 = pltpu.async_copy(path_v, path_hbm.at[b, 0], sem2)
        o1.wait()
        o2.wait()


@jax.jit
def _astar_sc(cm4, sqt, gm4, sm4):
    mesh = plsc.VectorSubcoreMesh(core_axis_name="c", subcore_axis_name="s",
                                  num_cores=NC, num_subcores=NS)
    f32 = jnp.float32
    run = pl.kernel(
        _astar_body,
        out_type=(jax.ShapeDtypeStruct((B, 1, H, W), f32),
                  jax.ShapeDtypeStruct((B, 1, H, W), jnp.int32)),
        mesh=mesh,
        compiler_params=pltpu.CompilerParams(needs_layout_passes=False),
        scratch_types=(
            pltpu.VMEM((H, W), f32),
            pltpu.VMEM((SQT,), f32),
            pltpu.VMEM((H, W), f32),
            pltpu.VMEM((H, W), f32),
            pltpu.VMEM((HW,), f32),
            pltpu.VMEM((HW,), f32),
            pltpu.VMEM((HW,), f32),
            pltpu.VMEM((H, W), f32),
            pltpu.VMEM((HW,), jnp.int32),
            pltpu.VMEM((H, W), jnp.int32),
            pltpu.SemaphoreType.DMA,
            pltpu.SemaphoreType.DMA,
            pltpu.SemaphoreType.DMA,
            pltpu.SemaphoreType.DMA,
        ),
    )
    return run(cm4, sqt, gm4, sm4)


def kernel(cost_maps, start_maps, goal_maps, heuristic_maps, obstacles_maps):
    anti = cost_maps[0, 0, 0, 0] * 0.0
    sqt = jnp.sqrt(jnp.arange(SQT, dtype=jnp.float32) + anti)
    return _astar_sc(cost_maps, sqt, goal_maps, start_maps)


# speedup vs baseline: 1.0551x; 1.0551x over previous
import math

import jax
import jax.numpy as jnp
from jax import lax
from jax.experimental import pallas as pl
from jax.experimental.pallas import tpu as pltpu
from jax.experimental.pallas import tpu_sc as plsc

B, H, W = 8, 32, 32
HW = H * W
TMAX = HW
SQW = math.sqrt(W)
NC, NS = 1, 16
L = 16
CPR = W // L
SQT = 1928


def _astar_body(cm_hbm, sqt_hbm, gm_hbm, sm_hbm, hist_hbm, path_hbm,
                cm_v, sqt_v, gm_v, sm_v, val_v, g_v, open_v, hist_v,
                parents_v, path_v, sem1, sem2, sem3, sem4):
    wid = lax.axis_index("s") * NC + lax.axis_index("c")

    @pl.when(wid < B)
    def _():
        b = wid
        c1 = pltpu.async_copy(cm_hbm.at[b, 0], cm_v, sem1)
        c2 = pltpu.async_copy(sqt_hbm, sqt_v, sem2)
        c3 = pltpu.async_copy(gm_hbm.at[b, 0], gm_v, sem3)
        c4 = pltpu.async_copy(sm_hbm.at[b, 0], sm_v, sem4)

        lane = lax.iota(jnp.int32, L)
        zero_f = jnp.zeros((L,), jnp.float32)
        one_f = jnp.ones((L,), jnp.float32)
        one_i = jnp.ones((L,), jnp.int32)
        lane0 = lane == 0

        c3.wait()
        c4.wait()

        def find_chunk(c, acc):
            r = c >> 1
            half = (c & 1) * L
            gmc = gm_v[r, pl.ds(half, L)]
            smc = sm_v[r, pl.ds(half, L)]
            negidx = -(c * L + lane).astype(jnp.float32)
            gacc, sacc = acc
            gcand = jnp.max(jnp.where(gmc > 0.5, negidx, -float(HW)))
            scand = jnp.max(jnp.where(smc > 0.5, negidx, -float(HW)))
            return jnp.maximum(gacc, gcand), jnp.maximum(sacc, scand)

        gneg, sneg = lax.fori_loop(0, HW // L, find_chunk,
                                   (jnp.float32(-HW), jnp.float32(-HW)))
        goal_i = (-gneg).astype(jnp.int32)
        start_i = (-sneg).astype(jnp.int32)

        goal_fill = jnp.full((L,), goal_i, jnp.int32)
        zero_i = jnp.zeros((L,), jnp.int32)

        def init_chunk(c, _):
            s = c * L
            g_v[pl.ds(s, L)] = zero_f
            open_v[pl.ds(s, L)] = zero_f
            val_v[pl.ds(s, L)] = zero_f
            parents_v[pl.ds(s, L)] = goal_fill
            hist_v[c >> 1, pl.ds((c & 1) * L, L)] = zero_f
            path_v[c >> 1, pl.ds((c & 1) * L, L)] = zero_i
            return 0

        lax.fori_loop(0, HW // L, init_chunk, 0)

        startv = jnp.full((L,), start_i, jnp.int32)
        goalv = jnp.full((L,), goal_i, jnp.int32)
        gyv = goalv >> 5
        gxv = goalv & 31
        c2.wait()

        def h_at(yv, xv):
            dyv = jnp.abs(yv - gyv)
            dxv = jnp.abs(xv - gxv)
            euc = plsc.load_gather(sqt_v, [dyv * dyv + dxv * dxv])
            cheb = jnp.maximum(dyv, dxv).astype(jnp.float32)
            return cheb + 0.001 * euc

        hs = h_at(startv >> 5, startv & 31)
        plsc.store_scatter(open_v, [startv], one_f, mask=lane0)
        plsc.store_scatter(val_v, [startv], jnp.exp(-(0.5 * hs) / SQW),
                           mask=lane0)
        plsc.store_scatter(path_v, [gyv, gxv], one_i, mask=lane0)
        c1.wait()

        nk = lane + (lane >= 4).astype(jnp.int32)
        dy = nk // 3 - 1
        dx = nk % 3 - 1
        nbr_lane = lane < 8

        def cond(carry):
            t, solved = carry[0], carry[1]
            return jnp.logical_and(t < TMAX, jnp.logical_not(solved))

        def body(carry):
            t, _, t1, ymin, ymax = carry

            def acond(st):
                return st[0] <= ymax

            def arow(st):
                r, bv, bi = st
                s = r * W
                v = val_v[pl.ds(s, L)]
                m = v > bv
                bv = jnp.where(m, v, bv)
                bi = jnp.where(m, s + lane, bi)
                v2 = val_v[pl.ds(s + L, L)]
                m2 = v2 > bv
                return (r + 1, jnp.where(m2, v2, bv),
                        jnp.where(m2, s + L + lane, bi))

            _, bv, bi = lax.while_loop(
                acond, arow,
                (ymin, jnp.full((L,), -1.0, jnp.float32),
                 jnp.zeros((L,), jnp.int32)))
            maxv = jnp.max(bv)
            nmax = jnp.max(jnp.where(bv == maxv, -bi.astype(jnp.float32),
                                     -float(HW)))
            indv = (-jnp.full((L,), nmax)).astype(jnp.int32)
            ind = (-nmax).astype(jnp.int32)
            solved = ind == goal_i

            iyv = indv >> 5
            ixv = indv & 31
            plsc.store_scatter(hist_v, [iyv, ixv], one_f, mask=lane0)
            rm = lane0 & (indv != goalv)
            plsc.store_scatter(open_v, [indv], zero_f, mask=rm)
            plsc.store_scatter(val_v, [indv], zero_f, mask=rm)

            new_gv = plsc.load_gather(g_v, [indv]) + plsc.load_gather(
                cm_v, [iyv, ixv])

            iy = ind >> 5
            ny = iyv + dy
            nx = ixv + dx
            valid = (nbr_lane & (ny >= 0) & (ny < H) & (nx >= 0) & (nx < W))
            nyc = jnp.where(valid, ny, 0)
            nxc = jnp.where(valid, nx, 0)
            nidx = nyc * W + nxc
            gn = plsc.load_gather(g_v, [nidx], mask=valid)
            on = plsc.load_gather(open_v, [nidx], mask=valid)
            hn = plsc.load_gather(hist_v, [nyc, nxc], mask=valid)
            hh = h_at(nyc, nxc)
            upd = valid & (((on <= 0.5) & (hn <= 0.5))
                           | ((on > 0.5) & (gn > new_gv)))
            plsc.store_scatter(g_v, [nidx], new_gv, mask=upd)
            plsc.store_scatter(open_v, [nidx], one_f, mask=upd)
            plsc.store_scatter(parents_v, [nidx], indv, mask=upd)
            fn = 0.5 * new_gv + 0.5 * hh
            plsc.store_scatter(val_v, [nidx], jnp.exp(-fn / SQW), mask=upd)

            t1 = jnp.where(solved, t, t1)
            ymin = jnp.minimum(ymin, jnp.maximum(iy - 1, 0))
            ymax = jnp.maximum(ymax, jnp.minimum(iy + 1, H - 1))
            return t + 1, solved, t1, ymin, ymax

        sy = start_i >> 5
        _, _, t1, _, _ = lax.while_loop(
            cond, body,
            (jnp.int32(0), jnp.bool_(False), jnp.int32(TMAX - 1), sy, sy))

        loc0 = jnp.max(plsc.load_gather(parents_v,
                                        [goalv]).astype(jnp.float32))

        def bt(_, loc):
            locv = jnp.full((L,), loc.astype(jnp.int32), jnp.int32)
            plsc.store_scatter(path_v, [locv >> 5, locv & 31], one_i,
                               mask=lane0)
            return jnp.max(plsc.load_gather(parents_v,
                                            [locv]).astype(jnp.float32))

        lax.fori_loop(0, t1, bt, loc0)

        o1 = pltpu.async_copy(hist_v, hist_hbm.at[b, 0], sem1)
        o2 = pltpu.async_copy(path_v, path_hbm.at[b, 0], sem2)
        o1.wait()
        o2.wait()


@jax.jit
def _astar_sc(cm4, sqt, gm4, sm4):
    mesh = plsc.VectorSubcoreMesh(core_axis_name="c", subcore_axis_name="s",
                                  num_cores=NC, num_subcores=NS)
    f32 = jnp.float32
    run = pl.kernel(
        _astar_body,
        out_type=(jax.ShapeDtypeStruct((B, 1, H, W), f32),
                  jax.ShapeDtypeStruct((B, 1, H, W), jnp.int32)),
        mesh=mesh,
        compiler_params=pltpu.CompilerParams(needs_layout_passes=False),
        scratch_types=(
            pltpu.VMEM((H, W), f32),
            pltpu.VMEM((SQT,), f32),
            pltpu.VMEM((H, W), f32),
            pltpu.VMEM((H, W), f32),
            pltpu.VMEM((HW,), f32),
            pltpu.VMEM((HW,), f32),
            pltpu.VMEM((HW,), f32),
            pltpu.VMEM((H, W), f32),
            pltpu.VMEM((HW,), jnp.int32),
            pltpu.VMEM((H, W), jnp.int32),
            pltpu.SemaphoreType.DMA,
            pltpu.SemaphoreType.DMA,
            pltpu.SemaphoreType.DMA,
            pltpu.SemaphoreType.DMA,
        ),
    )
    return run(cm4, sqt, gm4, sm4)


def kernel(cost_maps, start_maps, goal_maps, heuristic_maps, obstacles_maps):
    sqt = jnp.sqrt(lax.optimization_barrier(
        jnp.arange(SQT, dtype=jnp.float32)))
    return _astar_sc(cost_maps, sqt, goal_maps, start_maps)
